# baseline (speedup 1.0000x reference)
import functools
import math

import jax
import jax.numpy as jnp
from jax.experimental import pallas as pl
from jax.experimental.pallas import tpu as pltpu

_KT = 2048
_NT = 2048
_TP = 16


def _in_body(n_items, n_rows, x_ref, w_ref, t_ref, fvec_ref, csel_ref,
             ssel_ref, ew_ref, eb_ref, wt_ref, bi_ref, h_ref, ss_ref):
    k = pl.program_id(0)

    @pl.when(k == 0)
    def _init():
        h_ref[...] = jnp.zeros_like(h_ref)
        ss_ref[...] = jnp.zeros_like(ss_ref)

    xt = x_ref[...]
    col = k * _KT + jax.lax.broadcasted_iota(jnp.int32, xt.shape, 1)
    xt = jnp.where(col < n_items, xt, 0.0)
    wtile = w_ref[...]
    row = k * _KT + jax.lax.broadcasted_iota(jnp.int32, wtile.shape, 0)
    wtile = jnp.where(row < n_rows, wtile, 0.0)
    h_ref[...] += jnp.dot(xt, wtile, preferred_element_type=jnp.float32)
    ss_ref[...] += jnp.sum(xt * xt, axis=1, keepdims=True)

    @pl.when(k == pl.num_programs(0) - 1)
    def _finish():
        t = t_ref[...]
        temp = t * fvec_ref[...]
        te = jnp.cos(temp) * csel_ref[...] + jnp.sin(temp) * ssel_ref[...]
        emb = jnp.dot(te, ew_ref[...],
                      preferred_element_type=jnp.float32) + eb_ref[...]
        contrib = jnp.dot(emb, wt_ref[...],
                          preferred_element_type=jnp.float32)
        rn = jax.lax.rsqrt(jnp.maximum(ss_ref[...], 1e-24))
        h_ref[...] = jnp.tanh(h_ref[...] * rn + contrib + bi_ref[...])


def _out_body(h_ref, w_ref, b_ref, o_ref):
    o_ref[...] = jnp.dot(h_ref[...], w_ref[...],
                         preferred_element_type=jnp.float32) + b_ref[...]


def kernel(x, timesteps, emb_W, emb_b, W_in, b_in, W_out, b_out):
    B, N = x.shape
    H = W_in.shape[1]
    T = emb_W.shape[0]
    half = T // 2

    freqs = jnp.exp(-math.log(10000.0)
                    * jnp.arange(0, half, dtype=jnp.float32) / half)
    fvec = jnp.zeros((1, _TP), jnp.float32)
    fvec = fvec.at[0, :half].set(freqs).at[0, half:T].set(freqs)
    csel = jnp.zeros((1, _TP), jnp.float32).at[0, :half].set(1.0)
    ssel = jnp.zeros((1, _TP), jnp.float32).at[0, half:T].set(1.0)
    ew = jnp.zeros((_TP, _TP), jnp.float32).at[:T, :T].set(emb_W)
    eb = jnp.zeros((1, _TP), jnp.float32).at[0, :T].set(emb_b)
    wt = jnp.zeros((_TP, H), jnp.float32).at[:T, :].set(W_in[N:])
    tf = timesteps.astype(jnp.float32).reshape(B, 1)
    bi = b_in.reshape(1, H)
    bo = b_out.reshape(1, N)

    num_k = pl.cdiv(N, _KT)
    h = pl.pallas_call(
        functools.partial(_in_body, N, N + T),
        grid=(num_k,),
        in_specs=[
            pl.BlockSpec((B, _KT), lambda k: (0, k)),
            pl.BlockSpec((_KT, H), lambda k: (k, 0)),
            pl.BlockSpec((B, 1), lambda k: (0, 0)),
            pl.BlockSpec((1, _TP), lambda k: (0, 0)),
            pl.BlockSpec((1, _TP), lambda k: (0, 0)),
            pl.BlockSpec((1, _TP), lambda k: (0, 0)),
            pl.BlockSpec((_TP, _TP), lambda k: (0, 0)),
            pl.BlockSpec((1, _TP), lambda k: (0, 0)),
            pl.BlockSpec((_TP, H), lambda k: (0, 0)),
            pl.BlockSpec((1, H), lambda k: (0, 0)),
        ],
        out_specs=pl.BlockSpec((B, H), lambda k: (0, 0)),
        out_shape=jax.ShapeDtypeStruct((B, H), jnp.float32),
        scratch_shapes=[pltpu.VMEM((B, 1), jnp.float32)],
        compiler_params=pltpu.CompilerParams(
            dimension_semantics=("arbitrary",)),
    )(x, W_in, tf, fvec, csel, ssel, ew, eb, wt, bi)

    num_j = pl.cdiv(N, _NT)
    out = pl.pallas_call(
        _out_body,
        grid=(num_j,),
        in_specs=[
            pl.BlockSpec((B, H), lambda j: (0, 0)),
            pl.BlockSpec((H, _NT), lambda j: (0, j)),
            pl.BlockSpec((1, _NT), lambda j: (0, j)),
        ],
        out_specs=pl.BlockSpec((B, _NT), lambda j: (0, j)),
        out_shape=jax.ShapeDtypeStruct((B, N), jnp.float32),
        compiler_params=pltpu.CompilerParams(
            dimension_semantics=("parallel",)),
    )(h, W_out, bo)
    return out
